# Initial kernel scaffold; baseline (speedup 1.0000x reference)
import jax
import jax.numpy as jnp
from jax.experimental import pallas as pl


def kernel(x, patch_w, patch_b, cls_token, pos_embed, ln1_w, ln1_b, qkv_w, qkv_b, proj_w, proj_b, ln2_w, ln2_b, fc1_w, fc1_b, fc2_w, fc2_b, norm_w, norm_b, head_w, head_b):
    raise NotImplementedError("write your pallas kernel here")


# speedup vs baseline: 1.4600x; 1.4600x over previous
import math

import jax
import jax.numpy as jnp
from jax.experimental import pallas as pl
from jax.experimental.pallas import tpu as pltpu

DEPTH = 12
C = 768
H = 12
DH = 64
P = 16
IMG = 224
GRIDN = IMG // P
NP = GRIDN * GRIDN
KEEP_RATE = (1.0, 1.0, 1.0, 0.7, 1.0, 1.0, 0.7, 1.0, 1.0, 0.7, 1.0, 1.0)
SCALE = DH ** -0.5
F32 = jnp.float32


def _ln_rows(x, w, b, eps=1e-6):
    mu = jnp.mean(x, axis=-1, keepdims=True)
    xc = x - mu
    var = jnp.mean(xc * xc, axis=-1, keepdims=True)
    return xc * jax.lax.rsqrt(var + eps) * w + b


def _dot_nt(a, b):
    return jax.lax.dot_general(a, b, (((1,), (1,)), ((), ())),
                               preferred_element_type=F32)


def _mm(x, w, b, ln_w=None, ln_b=None, gelu=False, residual=None, bm=256):
    M, K = x.shape
    Nout = w.shape[0]
    has_ln = ln_w is not None
    has_res = residual is not None
    nm = pl.cdiv(M, bm)

    def kern(*refs):
        i = 3
        x_ref, w_ref, b_ref = refs[0], refs[1], refs[2]
        if has_ln:
            lw_ref, lb_ref = refs[i], refs[i + 1]
            i += 2
        if has_res:
            r_ref = refs[i]
            i += 1
        o_ref = refs[i]
        xb = x_ref[:, :]
        if has_ln:
            xb = _ln_rows(xb, lw_ref[0, :], lb_ref[0, :])
        acc = _dot_nt(xb, w_ref[:, :]) + b_ref[0, :]
        if gelu:
            acc = 0.5 * acc * (1.0 + jax.lax.erf(acc * (2.0 ** -0.5)))
        if has_res:
            acc = acc + r_ref[:, :]
        o_ref[:, :] = acc

    in_specs = [
        pl.BlockSpec((bm, K), lambda i: (i, 0)),
        pl.BlockSpec((Nout, K), lambda i: (0, 0)),
        pl.BlockSpec((1, Nout), lambda i: (0, 0)),
    ]
    args = [x, w, b.reshape(1, Nout)]
    if has_ln:
        in_specs += [pl.BlockSpec((1, K), lambda i: (0, 0)),
                     pl.BlockSpec((1, K), lambda i: (0, 0))]
        args += [ln_w.reshape(1, K), ln_b.reshape(1, K)]
    if has_res:
        in_specs.append(pl.BlockSpec((bm, Nout), lambda i: (i, 0)))
        args.append(residual)
    return pl.pallas_call(
        kern,
        grid=(nm,),
        in_specs=in_specs,
        out_specs=pl.BlockSpec((bm, Nout), lambda i: (i, 0)),
        out_shape=jax.ShapeDtypeStruct((M, Nout), F32),
        compiler_params=pltpu.CompilerParams(
            dimension_semantics=("parallel",)),
    )(*args)


def _patch_embed(xp, w, b, pos):
    B = xp.shape[0]

    def kern(x_ref, w_ref, b_ref, p_ref, o_ref):
        o_ref[0] = _dot_nt(x_ref[0], w_ref[:, :]) + b_ref[0, :] + p_ref[:, :]

    return pl.pallas_call(
        kern,
        grid=(B,),
        in_specs=[
            pl.BlockSpec((1, NP, C), lambda i: (i, 0, 0)),
            pl.BlockSpec((C, C), lambda i: (0, 0)),
            pl.BlockSpec((1, C), lambda i: (0, 0)),
            pl.BlockSpec((NP, C), lambda i: (0, 0)),
        ],
        out_specs=pl.BlockSpec((1, NP, C), lambda i: (i, 0, 0)),
        out_shape=jax.ShapeDtypeStruct((B, NP, C), F32),
        compiler_params=pltpu.CompilerParams(
            dimension_semantics=("parallel",)),
    )(xp, w, b.reshape(1, C), pos)


def _attention(qkv, n):
    B = qkv.shape[0]

    def kern(qkv_ref, o_ref, ca_ref):
        acc = jnp.zeros((1, n - 1), F32)
        for h in range(H):
            q = qkv_ref[0, :, h * DH:(h + 1) * DH]
            k = qkv_ref[0, :, C + h * DH:C + (h + 1) * DH]
            v = qkv_ref[0, :, 2 * C + h * DH:2 * C + (h + 1) * DH]
            s = _dot_nt(q, k) * SCALE
            m = jnp.max(s, axis=-1, keepdims=True)
            e = jnp.exp(s - m)
            p = e / jnp.sum(e, axis=-1, keepdims=True)
            o_ref[0, :, h * DH:(h + 1) * DH] = jnp.dot(
                p, v, preferred_element_type=F32)
            acc = acc + p[0:1, 1:]
        ca_ref[0, 0:1, :] = acc * (1.0 / H)

    return pl.pallas_call(
        kern,
        grid=(B,),
        in_specs=[pl.BlockSpec((1, n, 3 * C), lambda i: (i, 0, 0))],
        out_specs=[
            pl.BlockSpec((1, n, C), lambda i: (i, 0, 0)),
            pl.BlockSpec((1, 1, n - 1), lambda i: (i, 0, 0)),
        ],
        out_shape=[
            jax.ShapeDtypeStruct((B, n, C), F32),
            jax.ShapeDtypeStruct((B, 1, n - 1), F32),
        ],
        compiler_params=pltpu.CompilerParams(
            dimension_semantics=("parallel",)),
    )(qkv)


def _prune(h, cls_attn, kkeep):
    B, n, _ = h.shape
    nm = n - 1
    nout = kkeep + 2

    def kern(h_ref, ca_ref, o_ref):
        s_row = ca_ref[0, 0:1, :]
        s_col = jnp.reshape(s_row, (nm, 1))
        ii = jax.lax.broadcasted_iota(jnp.int32, (nm, nm), 0)
        jj = jax.lax.broadcasted_iota(jnp.int32, (nm, nm), 1)
        cond = (s_col > s_row) | ((s_col == s_row) & (ii < jj))
        rank = jnp.sum(cond.astype(F32), axis=0, keepdims=True)
        keep = (rank < float(kkeep)).astype(F32)
        lt = (ii <= jj).astype(F32)
        pos = jnp.dot(keep, lt, preferred_element_type=F32) - 1.0
        rows = jax.lax.broadcasted_iota(jnp.int32, (kkeep, nm), 0)
        g = ((rows.astype(F32) == pos) & (keep > 0.5)).astype(F32)
        hn = h_ref[0, 1:, :]
        o_ref[0, 0:1, :] = h_ref[0, 0:1, :]
        o_ref[0, 1:1 + kkeep, :] = jnp.dot(
            g, hn, preferred_element_type=F32)
        w_nt = s_row * (1.0 - keep)
        o_ref[0, 1 + kkeep:, :] = jnp.dot(
            w_nt, hn, preferred_element_type=F32)

    return pl.pallas_call(
        kern,
        grid=(B,),
        in_specs=[
            pl.BlockSpec((1, n, C), lambda i: (i, 0, 0)),
            pl.BlockSpec((1, 1, nm), lambda i: (i, 0, 0)),
        ],
        out_specs=pl.BlockSpec((1, nout, C), lambda i: (i, 0, 0)),
        out_shape=jax.ShapeDtypeStruct((B, nout, C), F32),
        compiler_params=pltpu.CompilerParams(
            dimension_semantics=("parallel",)),
    )(h, cls_attn)


def kernel(x, patch_w, patch_b, cls_token, pos_embed, ln1_w, ln1_b, qkv_w,
           qkv_b, proj_w, proj_b, ln2_w, ln2_b, fc1_w, fc1_b, fc2_w, fc2_b,
           norm_w, norm_b, head_w, head_b):
    B = x.shape[0]
    xp = x.reshape(B, 3, GRIDN, P, GRIDN, P).transpose(0, 2, 4, 1, 3, 5)
    xp = xp.reshape(B, NP, 3 * P * P)
    tok = _patch_embed(xp, patch_w.reshape(C, 3 * P * P), patch_b,
                       pos_embed[0, 1:])
    cls0 = cls_token[0] + pos_embed[0, 0:1]
    h = jnp.concatenate(
        [jnp.broadcast_to(cls0[None], (B, 1, C)), tok], axis=1)
    n = NP + 1
    for i in range(DEPTH):
        hf = h.reshape(B * n, C)
        qkv = _mm(hf, qkv_w[i], qkv_b[i], ln_w=ln1_w[i], ln_b=ln1_b[i])
        out, cls_attn = _attention(qkv.reshape(B, n, 3 * C), n)
        h = _mm(out.reshape(B * n, C), proj_w[i], proj_b[i],
                residual=hf).reshape(B, n, C)
        if KEEP_RATE[i] < 1.0:
            kkeep = math.ceil(KEEP_RATE[i] * (n - 1))
            h = _prune(h, cls_attn, kkeep)
            n = kkeep + 2
        hf = h.reshape(B * n, C)
        m = _mm(hf, fc1_w[i], fc1_b[i], ln_w=ln2_w[i], ln_b=ln2_b[i],
                gelu=True)
        h = _mm(m, fc2_w[i], fc2_b[i], residual=hf).reshape(B, n, C)
    cls_final = h[:, 0, :]
    logits = _mm(cls_final, head_w, head_b, ln_w=norm_w, ln_b=norm_b, bm=32)
    return logits


# speedup vs baseline: 1.6835x; 1.1530x over previous
import math

import jax
import jax.numpy as jnp
from jax.experimental import pallas as pl
from jax.experimental.pallas import tpu as pltpu

DEPTH = 12
C = 768
H = 12
DH = 64
P = 16
IMG = 224
GRIDN = IMG // P
NP = GRIDN * GRIDN
KEEP_RATE = (1.0, 1.0, 1.0, 0.7, 1.0, 1.0, 0.7, 1.0, 1.0, 0.7, 1.0, 1.0)
SCALE = DH ** -0.5
F32 = jnp.float32


def _ln_rows(x, w, b, eps=1e-6):
    mu = jnp.mean(x, axis=-1, keepdims=True)
    xc = x - mu
    var = jnp.mean(xc * xc, axis=-1, keepdims=True)
    return xc / jnp.sqrt(var + eps) * w + b


PREC = jax.lax.Precision.DEFAULT


def _dot_nt(a, b):
    return jax.lax.dot_general(a.astype(jnp.bfloat16), b.astype(jnp.bfloat16),
                               (((1,), (1,)), ((), ())),
                               preferred_element_type=F32, precision=PREC)


def _mm(x, w, b, ln_w=None, ln_b=None, gelu=False, residual=None, bm=256):
    M, K = x.shape
    Nout = w.shape[0]
    has_ln = ln_w is not None
    has_res = residual is not None
    nm = pl.cdiv(M, bm)

    def kern(*refs):
        i = 3
        x_ref, w_ref, b_ref = refs[0], refs[1], refs[2]
        if has_ln:
            lw_ref, lb_ref = refs[i], refs[i + 1]
            i += 2
        if has_res:
            r_ref = refs[i]
            i += 1
        o_ref = refs[i]
        xb = x_ref[:, :]
        if has_ln:
            xb = _ln_rows(xb, lw_ref[0, :], lb_ref[0, :])
        acc = _dot_nt(xb, w_ref[:, :]) + b_ref[0, :]
        if gelu:
            acc = 0.5 * acc * (1.0 + jax.lax.erf(acc * (2.0 ** -0.5)))
        if has_res:
            acc = acc + r_ref[:, :]
        o_ref[:, :] = acc

    in_specs = [
        pl.BlockSpec((bm, K), lambda i: (i, 0)),
        pl.BlockSpec((Nout, K), lambda i: (0, 0)),
        pl.BlockSpec((1, Nout), lambda i: (0, 0)),
    ]
    args = [x, w, b.reshape(1, Nout)]
    if has_ln:
        in_specs += [pl.BlockSpec((1, K), lambda i: (0, 0)),
                     pl.BlockSpec((1, K), lambda i: (0, 0))]
        args += [ln_w.reshape(1, K), ln_b.reshape(1, K)]
    if has_res:
        in_specs.append(pl.BlockSpec((bm, Nout), lambda i: (i, 0)))
        args.append(residual)
    return pl.pallas_call(
        kern,
        grid=(nm,),
        in_specs=in_specs,
        out_specs=pl.BlockSpec((bm, Nout), lambda i: (i, 0)),
        out_shape=jax.ShapeDtypeStruct((M, Nout), F32),
        compiler_params=pltpu.CompilerParams(
            dimension_semantics=("parallel",)),
    )(*args)


def _patch_embed(xp, w, b, pos):
    B = xp.shape[0]

    def kern(x_ref, w_ref, b_ref, p_ref, o_ref):
        o_ref[0] = _dot_nt(x_ref[0], w_ref[:, :]) + b_ref[0, :] + p_ref[:, :]

    return pl.pallas_call(
        kern,
        grid=(B,),
        in_specs=[
            pl.BlockSpec((1, NP, C), lambda i: (i, 0, 0)),
            pl.BlockSpec((C, C), lambda i: (0, 0)),
            pl.BlockSpec((1, C), lambda i: (0, 0)),
            pl.BlockSpec((NP, C), lambda i: (0, 0)),
        ],
        out_specs=pl.BlockSpec((1, NP, C), lambda i: (i, 0, 0)),
        out_shape=jax.ShapeDtypeStruct((B, NP, C), F32),
        compiler_params=pltpu.CompilerParams(
            dimension_semantics=("parallel",)),
    )(xp, w, b.reshape(1, C), pos)


def _attn_block(h, ln_w, ln_b, wqkv, bqkv, wproj, bproj, n):
    B = h.shape[0]

    def kern(h_ref, lw_ref, lb_ref, wq_ref, bq_ref, wp_ref, bp_ref,
             o_ref, ca_ref):
        hb = h_ref[0]
        y = _ln_rows(hb, lw_ref[0, :], lb_ref[0, :])
        qkv = _dot_nt(y, wq_ref[:, :]) + bq_ref[0, :]
        acc = jnp.zeros((1, n - 1), F32)
        outs = []
        for hh in range(H):
            q = qkv[:, hh * DH:(hh + 1) * DH]
            k = qkv[:, C + hh * DH:C + (hh + 1) * DH]
            v = qkv[:, 2 * C + hh * DH:2 * C + (hh + 1) * DH]
            s = _dot_nt(q, k) * SCALE
            m = jnp.max(s, axis=-1, keepdims=True)
            e = jnp.exp(s - m)
            p = e / jnp.sum(e, axis=-1, keepdims=True)
            outs.append(jnp.dot(p, v, preferred_element_type=F32,
                                precision=PREC))
            acc = acc + p[0:1, 1:]
        out = jnp.concatenate(outs, axis=1)
        o_ref[0] = hb + _dot_nt(out, wp_ref[:, :]) + bp_ref[0, :]
        ca_ref[0, 0:1, :] = acc * (1.0 / H)

    return pl.pallas_call(
        kern,
        grid=(B,),
        in_specs=[
            pl.BlockSpec((1, n, C), lambda i: (i, 0, 0)),
            pl.BlockSpec((1, C), lambda i: (0, 0)),
            pl.BlockSpec((1, C), lambda i: (0, 0)),
            pl.BlockSpec((3 * C, C), lambda i: (0, 0)),
            pl.BlockSpec((1, 3 * C), lambda i: (0, 0)),
            pl.BlockSpec((C, C), lambda i: (0, 0)),
            pl.BlockSpec((1, C), lambda i: (0, 0)),
        ],
        out_specs=[
            pl.BlockSpec((1, n, C), lambda i: (i, 0, 0)),
            pl.BlockSpec((1, 1, n - 1), lambda i: (i, 0, 0)),
        ],
        out_shape=[
            jax.ShapeDtypeStruct((B, n, C), F32),
            jax.ShapeDtypeStruct((B, 1, n - 1), F32),
        ],
        compiler_params=pltpu.CompilerParams(
            dimension_semantics=("parallel",)),
    )(h, ln_w.reshape(1, C), ln_b.reshape(1, C), wqkv,
      bqkv.reshape(1, 3 * C), wproj, bproj.reshape(1, C))


def _mlp_block(x, ln_w, ln_b, w1, b1, w2, b2, bm=256):
    M = x.shape[0]
    nm = pl.cdiv(M, bm)

    def kern(x_ref, lw_ref, lb_ref, w1_ref, b1_ref, w2_ref, b2_ref, o_ref):
        xb = x_ref[:, :]
        y = _ln_rows(xb, lw_ref[0, :], lb_ref[0, :])
        mid = _dot_nt(y, w1_ref[:, :]) + b1_ref[0, :]
        mid = 0.5 * mid * (1.0 + jax.lax.erf(mid * (2.0 ** -0.5)))
        o_ref[:, :] = xb + _dot_nt(mid, w2_ref[:, :]) + b2_ref[0, :]

    return pl.pallas_call(
        kern,
        grid=(nm,),
        in_specs=[
            pl.BlockSpec((bm, C), lambda i: (i, 0)),
            pl.BlockSpec((1, C), lambda i: (0, 0)),
            pl.BlockSpec((1, C), lambda i: (0, 0)),
            pl.BlockSpec((4 * C, C), lambda i: (0, 0)),
            pl.BlockSpec((1, 4 * C), lambda i: (0, 0)),
            pl.BlockSpec((C, 4 * C), lambda i: (0, 0)),
            pl.BlockSpec((1, C), lambda i: (0, 0)),
        ],
        out_specs=pl.BlockSpec((bm, C), lambda i: (i, 0)),
        out_shape=jax.ShapeDtypeStruct((M, C), F32),
        compiler_params=pltpu.CompilerParams(
            dimension_semantics=("parallel",)),
    )(x, ln_w.reshape(1, C), ln_b.reshape(1, C), w1, b1.reshape(1, 4 * C),
      w2, b2.reshape(1, C))


def _prune(h, cls_attn, kkeep):
    B, n, _ = h.shape
    nm = n - 1
    nout = kkeep + 2

    def kern(h_ref, ca_ref, o_ref):
        s_row = ca_ref[0, 0:1, :]
        s_col = jnp.reshape(s_row, (nm, 1))
        ii = jax.lax.broadcasted_iota(jnp.int32, (nm, nm), 0)
        jj = jax.lax.broadcasted_iota(jnp.int32, (nm, nm), 1)
        cond = (s_col > s_row) | ((s_col == s_row) & (ii < jj))
        rank = jnp.sum(cond.astype(F32), axis=0, keepdims=True)
        keep = (rank < float(kkeep)).astype(F32)
        lt = (ii <= jj).astype(F32)
        pos = jnp.dot(keep, lt, preferred_element_type=F32) - 1.0
        rows = jax.lax.broadcasted_iota(jnp.int32, (kkeep, nm), 0)
        g = ((rows.astype(F32) == pos) & (keep > 0.5)).astype(F32)
        hn = h_ref[0, 1:, :]
        o_ref[0, 0:1, :] = h_ref[0, 0:1, :]
        o_ref[0, 1:1 + kkeep, :] = jnp.dot(
            g, hn, preferred_element_type=F32, precision=PREC)
        w_nt = s_row * (1.0 - keep)
        o_ref[0, 1 + kkeep:, :] = jnp.dot(
            w_nt, hn, preferred_element_type=F32, precision=PREC)

    return pl.pallas_call(
        kern,
        grid=(B,),
        in_specs=[
            pl.BlockSpec((1, n, C), lambda i: (i, 0, 0)),
            pl.BlockSpec((1, 1, nm), lambda i: (i, 0, 0)),
        ],
        out_specs=pl.BlockSpec((1, nout, C), lambda i: (i, 0, 0)),
        out_shape=jax.ShapeDtypeStruct((B, nout, C), F32),
        compiler_params=pltpu.CompilerParams(
            dimension_semantics=("parallel",)),
    )(h, cls_attn)


def kernel(x, patch_w, patch_b, cls_token, pos_embed, ln1_w, ln1_b, qkv_w,
           qkv_b, proj_w, proj_b, ln2_w, ln2_b, fc1_w, fc1_b, fc2_w, fc2_b,
           norm_w, norm_b, head_w, head_b):
    B = x.shape[0]
    xp = x.reshape(B, 3, GRIDN, P, GRIDN, P).transpose(0, 2, 4, 1, 3, 5)
    xp = xp.reshape(B, NP, 3 * P * P)
    tok = _patch_embed(xp, patch_w.reshape(C, 3 * P * P), patch_b,
                       pos_embed[0, 1:])
    cls0 = cls_token[0] + pos_embed[0, 0:1]
    h = jnp.concatenate(
        [jnp.broadcast_to(cls0[None], (B, 1, C)), tok], axis=1)
    n = NP + 1
    for i in range(DEPTH):
        h, cls_attn = _attn_block(h, ln1_w[i], ln1_b[i], qkv_w[i], qkv_b[i],
                                  proj_w[i], proj_b[i], n)
        if KEEP_RATE[i] < 1.0:
            kkeep = math.ceil(KEEP_RATE[i] * (n - 1))
            h = _prune(h, cls_attn, kkeep)
            n = kkeep + 2
        h = _mlp_block(h.reshape(B * n, C), ln2_w[i], ln2_b[i], fc1_w[i],
                       fc1_b[i], fc2_w[i], fc2_b[i]).reshape(B, n, C)
    cls_final = h[:, 0, :]
    logits = _mm(cls_final, head_w, head_b, ln_w=norm_w, ln_b=norm_b, bm=32)
    return logits
